# baseline (speedup 1.0000x reference)
import functools

import jax
import jax.numpy as jnp
from jax import lax
from jax.experimental import pallas as pl
from jax.experimental.pallas import tpu as pltpu
from jax.experimental.pallas import tpu_sc as plsc

LATENT_DIM = 256
CODEBOOK_SIZE = 8192
N_TOKENS = 8192
TN = 1024
TK = 2048
N_TILES = N_TOKENS // TN
K_TILES = CODEBOOK_SIZE // TK


_NCHUNK = TK // 128
_RB = 64


def _argmin_kernel(x_ref, w_ref, xsq_ref, wsq_ref, idx_ref, minv_ref,
                   bv_ref, bc_ref):
    j = pl.program_id(1)
    s2 = jax.lax.dot_general(
        x_ref[...] * -2.0, w_ref[...],
        (((1,), (0,)), ((), ())),
        preferred_element_type=jnp.float32,
    )
    xb = jnp.broadcast_to(xsq_ref[...], (TN, 128))
    wsq = wsq_ref[...]

    def _dchunk(k):
        return (xb + s2[:, k * 128:(k + 1) * 128]) + jnp.broadcast_to(
            wsq[:, k * 128:(k + 1) * 128], (TN, 128))

    def _tourney(bv, bc, ks):
        for k in ks:
            dk = _dchunk(k)
            better = dk < bv
            bv = jnp.where(better, dk, bv)
            bc = jnp.where(better, jnp.full((TN, 128), j * _NCHUNK + k,
                                            jnp.int32), bc)
        return bv, bc

    @pl.when(j == 0)
    def _seed():
        bv, bc = _tourney(_dchunk(0), jnp.zeros((TN, 128), jnp.int32),
                          range(1, _NCHUNK))
        bv_ref[...] = bv
        bc_ref[...] = bc

    @pl.when(j > 0)
    def _update():
        bv, bc = _tourney(bv_ref[...], bc_ref[...], range(_NCHUNK))
        bv_ref[...] = bv
        bc_ref[...] = bc

    @pl.when(j == K_TILES - 1)
    def _finalize():
        bv = bv_ref[...]
        gidx = bc_ref[...] * 128 + jax.lax.broadcasted_iota(
            jnp.int32, (TN, 128), 1)
        lm = jnp.min(bv, axis=1)
        li = jnp.min(jnp.where(bv == lm[:, None], gidx, jnp.int32(2**30)),
                     axis=1)
        minv_ref[0, 0, :] = lm
        idx_ref[0, 0, :] = li


def _distance_argmin(flat, codebook, xsq, wsq):
    n_tiles = flat.shape[0] // TN
    idx3, minv3 = pl.pallas_call(
        _argmin_kernel,
        grid=(n_tiles, K_TILES),
        in_specs=[
            pl.BlockSpec((TN, LATENT_DIM), lambda i, j: (i, 0)),
            pl.BlockSpec((LATENT_DIM, TK), lambda i, j: (0, j)),
            pl.BlockSpec((TN, 1), lambda i, j: (i, 0)),
            pl.BlockSpec((1, TK), lambda i, j: (0, j)),
        ],
        out_specs=[
            pl.BlockSpec((1, 1, TN), lambda i, j: (i, 0, 0)),
            pl.BlockSpec((1, 1, TN), lambda i, j: (i, 0, 0)),
        ],
        out_shape=[
            jax.ShapeDtypeStruct((n_tiles, 1, TN), jnp.int32),
            jax.ShapeDtypeStruct((n_tiles, 1, TN), jnp.float32),
        ],
        scratch_shapes=[
            pltpu.VMEM((TN, 128), jnp.float32),
            pltpu.VMEM((TN, 128), jnp.int32),
        ],
        compiler_params=pltpu.CompilerParams(
            dimension_semantics=("parallel", "arbitrary"),
        ),
    )(flat, codebook, xsq, wsq)
    return idx3.reshape(-1), minv3.reshape(-1)


_NC, _NS = 2, 16
_NW = _NC * _NS
_HCHUNK = CODEBOOK_SIZE // _NS

_SC_MESH = plsc.VectorSubcoreMesh(core_axis_name="c", subcore_axis_name="s")


def _make_sc_tail(n_tok):
    chunk = n_tok // _NW
    grows = chunk // 128
    idx_rows = n_tok // 128
    hrows = idx_rows // _NS
    lchunk = n_tok // _NS

    def _sc_tail(table_hbm, idx2_hbm, minv_hbm,
                 quant_hbm, counts_hbm, loss_hbm,
                 idx_g, idx_h, rows_v, ones_v, cnt_v, minv_v, acc_v,
                 counts_sh, sem):
        cid = lax.axis_index("c")
        sid = lax.axis_index("s")
        wid = sid * _NC + cid
        base = wid * chunk
        zero16 = jnp.zeros((16,), jnp.float32)
        ones16 = jnp.ones((16,), jnp.float32)

        pltpu.sync_copy(idx2_hbm.at[pl.ds(wid * grows, grows)], idx_g)
        for c in range(grows):
            pltpu.async_copy(table_hbm.at[idx_g.at[c]], rows_v, sem).wait()
            pltpu.sync_copy(rows_v, quant_hbm.at[pl.ds(base + c * 128, 128)])

        @pl.when(cid == 0)
        def _zero_counts():
            for i in range(_HCHUNK // 16):
                cnt_v[pl.ds(i * 16, 16)] = zero16
            pltpu.sync_copy(cnt_v, counts_sh.at[pl.ds(sid * _HCHUNK, _HCHUNK)])

        plsc.subcore_barrier()

        @pl.when(cid == 0)
        def _scatter_add():
            for i in range(128 // 16):
                ones_v[pl.ds(i * 16, 16)] = ones16
            pltpu.sync_copy(idx2_hbm.at[pl.ds(sid * hrows, hrows)], idx_h)
            for j in range(hrows):
                pltpu.sync_copy(ones_v, counts_sh.at[idx_h.at[j]], add=True)

        plsc.subcore_barrier()

        @pl.when(cid == 0)
        def _emit_counts():
            pltpu.sync_copy(counts_sh.at[pl.ds(sid * _HCHUNK, _HCHUNK)], cnt_v)
            pltpu.sync_copy(cnt_v, counts_hbm.at[pl.ds(sid * _HCHUNK, _HCHUNK)])

        @pl.when(cid == 0)
        def _loss_partial():
            pltpu.sync_copy(minv_hbm.at[pl.ds(sid * lchunk, lchunk)], minv_v)
            acc = zero16
            for i in range(lchunk // 16):
                acc = acc + minv_v[pl.ds(i * 16, 16)]
            acc_v[...] = acc
            pltpu.sync_copy(acc_v, loss_hbm.at[sid])

    return pl.kernel(
        _sc_tail,
        out_type=[
            jax.ShapeDtypeStruct((n_tok, LATENT_DIM), jnp.float32),
            jax.ShapeDtypeStruct((CODEBOOK_SIZE,), jnp.float32),
            jax.ShapeDtypeStruct((_NS, 16), jnp.float32),
        ],
        mesh=_SC_MESH,
        scratch_types=[
            pltpu.VMEM((grows, 128), jnp.int32),
            pltpu.VMEM((hrows, 128), jnp.int32),
            pltpu.VMEM((128, LATENT_DIM), jnp.float32),
            pltpu.VMEM((128,), jnp.float32),
            pltpu.VMEM((_HCHUNK,), jnp.float32),
            pltpu.VMEM((lchunk,), jnp.float32),
            pltpu.VMEM((16,), jnp.float32),
            pltpu.VMEM_SHARED((CODEBOOK_SIZE,), jnp.float32),
            pltpu.SemaphoreType.DMA,
        ],
    )


_HALF = N_TOKENS // 2
_sc_tail_half = _make_sc_tail(_HALF)


def kernel(z, codebook):
    commitment_cost = 1.0
    flat = jnp.reshape(z, (-1, LATENT_DIM))
    xsq = jnp.sum(flat ** 2, axis=-1)
    wsq2 = jnp.sum(codebook ** 2, axis=0).reshape(1, CODEBOOK_SIZE)
    table = codebook.T

    idxs, quants, counts, losses = [], [], [], []
    for h in range(2):
        fl = flat[h * _HALF:(h + 1) * _HALF]
        xs = xsq[h * _HALF:(h + 1) * _HALF]
        indices_h, minv_h = _distance_argmin(
            fl, codebook, xs.reshape(_HALF, 1), wsq2)
        q_h, c_h, l_h = _sc_tail_half(
            table, indices_h.reshape(_HALF // 128, 128), minv_h)
        idxs.append(indices_h)
        quants.append(q_h)
        counts.append(c_h)
        losses.append(l_h)

    indices = jnp.concatenate(idxs)
    quantised = jnp.concatenate(quants)
    avg_probs = (counts[0] + counts[1]) * (1.0 / N_TOKENS)
    commitment_loss = commitment_cost * (
        (jnp.sum(losses[0]) + jnp.sum(losses[1]))
        * (1.0 / (N_TOKENS * LATENT_DIM)))
    return (quantised, commitment_loss, avg_probs, indices)


# speedup vs baseline: 1.1074x; 1.1074x over previous
import functools

import jax
import jax.numpy as jnp
from jax import lax
from jax.experimental import pallas as pl
from jax.experimental.pallas import tpu as pltpu
from jax.experimental.pallas import tpu_sc as plsc

LATENT_DIM = 256
CODEBOOK_SIZE = 8192
N_TOKENS = 8192
TN = 1024
TK = 2048
N_TILES = N_TOKENS // TN
K_TILES = CODEBOOK_SIZE // TK


_NCHUNK = TK // 128
_RB = 64


def _argmin_kernel(x_ref, w_ref, xsq_ref, wsq_ref, idx_ref, minv_ref,
                   bv_ref, bc_ref):
    j = pl.program_id(1)
    s2 = jax.lax.dot_general(
        x_ref[...] * -2.0, w_ref[...],
        (((1,), (0,)), ((), ())),
        preferred_element_type=jnp.float32,
    )
    xb = jnp.broadcast_to(xsq_ref[...], (TN, 128))
    wsq = wsq_ref[...]

    def _dchunk(k):
        return (xb + s2[:, k * 128:(k + 1) * 128]) + jnp.broadcast_to(
            wsq[:, k * 128:(k + 1) * 128], (TN, 128))

    def _tourney(bv, bc, ks):
        for k in ks:
            dk = _dchunk(k)
            better = dk < bv
            bv = jnp.where(better, dk, bv)
            bc = jnp.where(better, jnp.full((TN, 128), j * _NCHUNK + k,
                                            jnp.int32), bc)
        return bv, bc

    @pl.when(j == 0)
    def _seed():
        bv, bc = _tourney(_dchunk(0), jnp.zeros((TN, 128), jnp.int32),
                          range(1, _NCHUNK))
        bv_ref[...] = bv
        bc_ref[...] = bc

    @pl.when(j > 0)
    def _update():
        bv, bc = _tourney(bv_ref[...], bc_ref[...], range(_NCHUNK))
        bv_ref[...] = bv
        bc_ref[...] = bc

    @pl.when(j == K_TILES - 1)
    def _finalize():
        bv = bv_ref[...]
        gidx = bc_ref[...] * 128 + jax.lax.broadcasted_iota(
            jnp.int32, (TN, 128), 1)
        lm = jnp.min(bv, axis=1)
        li = jnp.min(jnp.where(bv == lm[:, None], gidx, jnp.int32(2**30)),
                     axis=1)
        minv_ref[0, 0, :] = lm
        idx_ref[0, 0, :] = li


def _distance_argmin(flat, codebook, xsq, wsq):
    n_tiles = flat.shape[0] // TN
    idx3, minv3 = pl.pallas_call(
        _argmin_kernel,
        grid=(n_tiles, K_TILES),
        in_specs=[
            pl.BlockSpec((TN, LATENT_DIM), lambda i, j: (i, 0)),
            pl.BlockSpec((LATENT_DIM, TK), lambda i, j: (0, j)),
            pl.BlockSpec((TN, 1), lambda i, j: (i, 0)),
            pl.BlockSpec((1, TK), lambda i, j: (0, j)),
        ],
        out_specs=[
            pl.BlockSpec((1, 1, TN), lambda i, j: (i, 0, 0)),
            pl.BlockSpec((1, 1, TN), lambda i, j: (i, 0, 0)),
        ],
        out_shape=[
            jax.ShapeDtypeStruct((n_tiles, 1, TN), jnp.int32),
            jax.ShapeDtypeStruct((n_tiles, 1, TN), jnp.float32),
        ],
        scratch_shapes=[
            pltpu.VMEM((TN, 128), jnp.float32),
            pltpu.VMEM((TN, 128), jnp.int32),
        ],
        compiler_params=pltpu.CompilerParams(
            dimension_semantics=("parallel", "arbitrary"),
        ),
    )(flat, codebook, xsq, wsq)
    return idx3.reshape(-1), minv3.reshape(-1)


_NC, _NS = 2, 16
_NW = _NC * _NS
_HCHUNK = CODEBOOK_SIZE // _NS

_SC_MESH = plsc.VectorSubcoreMesh(core_axis_name="c", subcore_axis_name="s")


def _make_sc_tail(n_tok):
    chunk = n_tok // _NW
    grows = chunk // 128
    idx_rows = n_tok // 128
    hrows = idx_rows // _NS
    lchunk = n_tok // _NS

    def _sc_tail(table_hbm, idx2_hbm, minv_hbm,
                 quant_hbm, counts_hbm, loss_hbm,
                 idx_g, idx_h, rows_v, ones_v, cnt_v, minv_v, acc_v,
                 counts_sh, sem):
        cid = lax.axis_index("c")
        sid = lax.axis_index("s")
        wid = sid * _NC + cid
        base = wid * chunk
        zero16 = jnp.zeros((16,), jnp.float32)
        ones16 = jnp.ones((16,), jnp.float32)

        pltpu.sync_copy(idx2_hbm.at[pl.ds(wid * grows, grows)], idx_g)
        for c in range(grows):
            pltpu.async_copy(table_hbm.at[idx_g.at[c]], rows_v, sem).wait()
            pltpu.sync_copy(rows_v, quant_hbm.at[pl.ds(base + c * 128, 128)])

        @pl.when(cid == 0)
        def _zero_counts():
            for i in range(_HCHUNK // 16):
                cnt_v[pl.ds(i * 16, 16)] = zero16
            pltpu.sync_copy(cnt_v, counts_sh.at[pl.ds(sid * _HCHUNK, _HCHUNK)])

        plsc.subcore_barrier()

        @pl.when(cid == 0)
        def _scatter_add():
            for i in range(128 // 16):
                ones_v[pl.ds(i * 16, 16)] = ones16
            pltpu.sync_copy(idx2_hbm.at[pl.ds(sid * hrows, hrows)], idx_h)
            for j in range(hrows):
                pltpu.sync_copy(ones_v, counts_sh.at[idx_h.at[j]], add=True)

        plsc.subcore_barrier()

        @pl.when(cid == 0)
        def _emit_counts():
            pltpu.sync_copy(counts_sh.at[pl.ds(sid * _HCHUNK, _HCHUNK)], cnt_v)
            pltpu.sync_copy(cnt_v, counts_hbm.at[pl.ds(sid * _HCHUNK, _HCHUNK)])

        @pl.when(cid == 0)
        def _loss_partial():
            pltpu.sync_copy(minv_hbm.at[pl.ds(sid * lchunk, lchunk)], minv_v)
            acc = zero16
            for i in range(lchunk // 16):
                acc = acc + minv_v[pl.ds(i * 16, 16)]
            acc_v[...] = acc
            pltpu.sync_copy(acc_v, loss_hbm.at[sid])

    return pl.kernel(
        _sc_tail,
        out_type=[
            jax.ShapeDtypeStruct((n_tok, LATENT_DIM), jnp.float32),
            jax.ShapeDtypeStruct((CODEBOOK_SIZE,), jnp.float32),
            jax.ShapeDtypeStruct((_NS, 16), jnp.float32),
        ],
        mesh=_SC_MESH,
        scratch_types=[
            pltpu.VMEM((grows, 128), jnp.int32),
            pltpu.VMEM((hrows, 128), jnp.int32),
            pltpu.VMEM((128, LATENT_DIM), jnp.float32),
            pltpu.VMEM((128,), jnp.float32),
            pltpu.VMEM((_HCHUNK,), jnp.float32),
            pltpu.VMEM((lchunk,), jnp.float32),
            pltpu.VMEM((16,), jnp.float32),
            pltpu.VMEM_SHARED((CODEBOOK_SIZE,), jnp.float32),
            pltpu.SemaphoreType.DMA,
        ],
    )


_sc_tail_full = _make_sc_tail(N_TOKENS)


def kernel(z, codebook):
    commitment_cost = 1.0
    flat = jnp.reshape(z, (-1, LATENT_DIM))
    xsq = jnp.sum(flat ** 2, axis=-1)
    wsq = jnp.sum(codebook ** 2, axis=0)
    indices, minv = _distance_argmin(
        flat, codebook,
        xsq.reshape(N_TOKENS, 1), wsq.reshape(1, CODEBOOK_SIZE))
    table = codebook.T
    quantised, code_counts, loss_parts = _sc_tail_full(
        table, indices.reshape(N_TOKENS // 128, 128), minv)
    avg_probs = code_counts * (1.0 / N_TOKENS)
    commitment_loss = commitment_cost * (
        jnp.sum(loss_parts) * (1.0 / (N_TOKENS * LATENT_DIM)))
    return (quantised, commitment_loss, avg_probs, indices)


# speedup vs baseline: 1.1090x; 1.0015x over previous
import functools

import jax
import jax.numpy as jnp
from jax import lax
from jax.experimental import pallas as pl
from jax.experimental.pallas import tpu as pltpu
from jax.experimental.pallas import tpu_sc as plsc

LATENT_DIM = 256
CODEBOOK_SIZE = 8192
N_TOKENS = 8192
TN = 1024
TK = 2048
N_TILES = N_TOKENS // TN
K_TILES = CODEBOOK_SIZE // TK


_NCHUNK = TK // 128
_RB = 64


def _argmin_kernel(x_ref, w_ref, xsq_ref, wsq_ref, idx_ref, minv_ref,
                   bv_ref, bc_ref):
    j = pl.program_id(1)
    s2 = jax.lax.dot_general(
        x_ref[...] * -2.0, w_ref[...],
        (((1,), (0,)), ((), ())),
        preferred_element_type=jnp.float32,
    )
    xb = jnp.broadcast_to(xsq_ref[...], (TN, 128))
    wsq = wsq_ref[...]

    def _dchunk(k):
        return (xb + s2[:, k * 128:(k + 1) * 128]) + jnp.broadcast_to(
            wsq[:, k * 128:(k + 1) * 128], (TN, 128))

    def _tourney(bv, bc, ks):
        leaves = [(_dchunk(k),
                   jnp.full((TN, 128), j * _NCHUNK + k, jnp.int32))
                  for k in ks]
        while len(leaves) > 1:
            nxt = []
            for a in range(0, len(leaves) - 1, 2):
                va, ca = leaves[a]
                vb, cb = leaves[a + 1]
                better = vb < va
                nxt.append((jnp.where(better, vb, va),
                            jnp.where(better, cb, ca)))
            if len(leaves) % 2:
                nxt.append(leaves[-1])
            leaves = nxt
        dv, dc = leaves[0]
        better = dv < bv
        return jnp.where(better, dv, bv), jnp.where(better, dc, bc)

    @pl.when(j == 0)
    def _seed():
        bv, bc = _tourney(_dchunk(0), jnp.zeros((TN, 128), jnp.int32),
                          range(1, _NCHUNK))
        bv_ref[...] = bv
        bc_ref[...] = bc

    @pl.when(j > 0)
    def _update():
        bv, bc = _tourney(bv_ref[...], bc_ref[...], range(_NCHUNK))
        bv_ref[...] = bv
        bc_ref[...] = bc

    @pl.when(j == K_TILES - 1)
    def _finalize():
        bv = bv_ref[...]
        gidx = bc_ref[...] * 128 + jax.lax.broadcasted_iota(
            jnp.int32, (TN, 128), 1)
        lm = jnp.min(bv, axis=1)
        li = jnp.min(jnp.where(bv == lm[:, None], gidx, jnp.int32(2**30)),
                     axis=1)
        minv_ref[0, 0, :] = lm
        idx_ref[0, 0, :] = li


def _distance_argmin(flat, codebook, xsq, wsq):
    n_tiles = flat.shape[0] // TN
    idx3, minv3 = pl.pallas_call(
        _argmin_kernel,
        grid=(n_tiles, K_TILES),
        in_specs=[
            pl.BlockSpec((TN, LATENT_DIM), lambda i, j: (i, 0)),
            pl.BlockSpec((LATENT_DIM, TK), lambda i, j: (0, j)),
            pl.BlockSpec((TN, 1), lambda i, j: (i, 0)),
            pl.BlockSpec((1, TK), lambda i, j: (0, j)),
        ],
        out_specs=[
            pl.BlockSpec((1, 1, TN), lambda i, j: (i, 0, 0)),
            pl.BlockSpec((1, 1, TN), lambda i, j: (i, 0, 0)),
        ],
        out_shape=[
            jax.ShapeDtypeStruct((n_tiles, 1, TN), jnp.int32),
            jax.ShapeDtypeStruct((n_tiles, 1, TN), jnp.float32),
        ],
        scratch_shapes=[
            pltpu.VMEM((TN, 128), jnp.float32),
            pltpu.VMEM((TN, 128), jnp.int32),
        ],
        compiler_params=pltpu.CompilerParams(
            dimension_semantics=("parallel", "arbitrary"),
        ),
    )(flat, codebook, xsq, wsq)
    return idx3.reshape(-1), minv3.reshape(-1)


_NC, _NS = 2, 16
_NW = _NC * _NS
_HCHUNK = CODEBOOK_SIZE // _NS

_SC_MESH = plsc.VectorSubcoreMesh(core_axis_name="c", subcore_axis_name="s")


def _make_sc_tail(n_tok):
    chunk = n_tok // _NW
    grows = chunk // 128
    idx_rows = n_tok // 128
    hrows = idx_rows // _NS
    lchunk = n_tok // _NS

    def _sc_tail(table_hbm, idx2_hbm, minv_hbm,
                 quant_hbm, counts_hbm, loss_hbm,
                 idx_g, idx_h, rows_v, ones_v, cnt_v, minv_v, acc_v,
                 counts_sh, sem):
        cid = lax.axis_index("c")
        sid = lax.axis_index("s")
        wid = sid * _NC + cid
        base = wid * chunk
        zero16 = jnp.zeros((16,), jnp.float32)
        ones16 = jnp.ones((16,), jnp.float32)

        pltpu.sync_copy(idx2_hbm.at[pl.ds(wid * grows, grows)], idx_g)
        for c in range(grows):
            pltpu.async_copy(table_hbm.at[idx_g.at[c]], rows_v, sem).wait()
            pltpu.sync_copy(rows_v, quant_hbm.at[pl.ds(base + c * 128, 128)])

        @pl.when(cid == 0)
        def _zero_counts():
            for i in range(_HCHUNK // 16):
                cnt_v[pl.ds(i * 16, 16)] = zero16
            pltpu.sync_copy(cnt_v, counts_sh.at[pl.ds(sid * _HCHUNK, _HCHUNK)])

        plsc.subcore_barrier()

        @pl.when(cid == 0)
        def _scatter_add():
            for i in range(128 // 16):
                ones_v[pl.ds(i * 16, 16)] = ones16
            pltpu.sync_copy(idx2_hbm.at[pl.ds(sid * hrows, hrows)], idx_h)
            for j in range(hrows):
                pltpu.sync_copy(ones_v, counts_sh.at[idx_h.at[j]], add=True)

        plsc.subcore_barrier()

        @pl.when(cid == 0)
        def _emit_counts():
            pltpu.sync_copy(counts_sh.at[pl.ds(sid * _HCHUNK, _HCHUNK)], cnt_v)
            pltpu.sync_copy(cnt_v, counts_hbm.at[pl.ds(sid * _HCHUNK, _HCHUNK)])

        @pl.when(cid == 0)
        def _loss_partial():
            pltpu.sync_copy(minv_hbm.at[pl.ds(sid * lchunk, lchunk)], minv_v)
            acc = zero16
            for i in range(lchunk // 16):
                acc = acc + minv_v[pl.ds(i * 16, 16)]
            acc_v[...] = acc
            pltpu.sync_copy(acc_v, loss_hbm.at[sid])

    return pl.kernel(
        _sc_tail,
        out_type=[
            jax.ShapeDtypeStruct((n_tok, LATENT_DIM), jnp.float32),
            jax.ShapeDtypeStruct((CODEBOOK_SIZE,), jnp.float32),
            jax.ShapeDtypeStruct((_NS, 16), jnp.float32),
        ],
        mesh=_SC_MESH,
        scratch_types=[
            pltpu.VMEM((grows, 128), jnp.int32),
            pltpu.VMEM((hrows, 128), jnp.int32),
            pltpu.VMEM((128, LATENT_DIM), jnp.float32),
            pltpu.VMEM((128,), jnp.float32),
            pltpu.VMEM((_HCHUNK,), jnp.float32),
            pltpu.VMEM((lchunk,), jnp.float32),
            pltpu.VMEM((16,), jnp.float32),
            pltpu.VMEM_SHARED((CODEBOOK_SIZE,), jnp.float32),
            pltpu.SemaphoreType.DMA,
        ],
    )


_sc_tail_full = _make_sc_tail(N_TOKENS)


def kernel(z, codebook):
    commitment_cost = 1.0
    flat = jnp.reshape(z, (-1, LATENT_DIM))
    xsq = jnp.sum(flat ** 2, axis=-1)
    wsq = jnp.sum(codebook ** 2, axis=0)
    indices, minv = _distance_argmin(
        flat, codebook,
        xsq.reshape(N_TOKENS, 1), wsq.reshape(1, CODEBOOK_SIZE))
    table = codebook.T
    quantised, code_counts, loss_parts = _sc_tail_full(
        table, indices.reshape(N_TOKENS // 128, 128), minv)
    avg_probs = code_counts * (1.0 / N_TOKENS)
    commitment_loss = commitment_cost * (
        jnp.sum(loss_parts) * (1.0 / (N_TOKENS * LATENT_DIM)))
    return (quantised, commitment_loss, avg_probs, indices)
